# Initial kernel scaffold; baseline (speedup 1.0000x reference)
import functools

import jax
import jax.numpy as jnp
from jax import lax
from jax.experimental import pallas as pl
from jax.experimental.pallas import tpu as pltpu
from jax.experimental.pallas import tpu_sc as plsc

EMB_D = 32


def _make_gather(num_rows: int):
    info = plsc.get_sparse_core_info()
    nw = info.num_cores * info.num_subcores
    rows_per_w = num_rows // nw
    chunk = 1024
    n_chunks = rows_per_w // chunk
    mesh = plsc.VectorSubcoreMesh(core_axis_name="c", subcore_axis_name="s")

    @functools.partial(
        pl.kernel,
        mesh=mesh,
        out_type=jax.ShapeDtypeStruct((num_rows, EMB_D), jnp.float32),
        scratch_types=[
            pltpu.VMEM((chunk,), jnp.int32),
            pltpu.VMEM((chunk, EMB_D), jnp.float32),
            pltpu.SemaphoreType.DMA,
        ],
    )
    def gather(idx_hbm, table_hbm, out_hbm, idx_v, rows_v, sem):
        wid = lax.axis_index("s") * info.num_cores + lax.axis_index("c")
        base = wid * rows_per_w

        def body(i, carry):
            off = base + i * chunk
            pltpu.sync_copy(idx_hbm.at[pl.ds(off, chunk)], idx_v)
            pltpu.async_copy(table_hbm.at[idx_v], rows_v, sem).wait()
            pltpu.sync_copy(rows_v, out_hbm.at[pl.ds(off, chunk)])
            return carry

        lax.fori_loop(0, n_chunks, body, 0)

    return gather


def kernel(IX, weight):
    b0, b1 = IX.shape
    num_rows = b0 * b1
    flat_ix = IX.reshape(num_rows).astype(jnp.int32)
    out = _make_gather(num_rows)(flat_ix, weight)
    return out.reshape(b0, b1, EMB_D)


# speedup vs baseline: 1.0954x; 1.0954x over previous
import functools

import jax
import jax.numpy as jnp
from jax import lax
from jax.experimental import pallas as pl
from jax.experimental.pallas import tpu as pltpu
from jax.experimental.pallas import tpu_sc as plsc

EMB_D = 32


def _make_gather(num_rows: int):
    info = plsc.get_sparse_core_info()
    nw = info.num_cores * info.num_subcores
    rows_per_w = num_rows // nw
    chunk = 1024
    n_chunks = rows_per_w // chunk
    mesh = plsc.VectorSubcoreMesh(core_axis_name="c", subcore_axis_name="s")

    @functools.partial(
        pl.kernel,
        mesh=mesh,
        out_type=jax.ShapeDtypeStruct((num_rows, EMB_D), jnp.float32),
        scratch_types=[
            pltpu.VMEM((chunk,), jnp.int32),
            pltpu.VMEM((chunk, EMB_D), jnp.float32),
            pltpu.SemaphoreType.DMA,
        ],
        compiler_params=pltpu.CompilerParams(use_tc_tiling_on_sc=False),
    )
    def gather(idx_hbm, table_hbm, out_hbm, idx_v, rows_v, sem):
        wid = lax.axis_index("s") * info.num_cores + lax.axis_index("c")
        base = wid * rows_per_w

        def body(i, carry):
            off = base + i * chunk
            pltpu.sync_copy(idx_hbm.at[pl.ds(off, chunk)], idx_v)
            pltpu.async_copy(table_hbm.at[idx_v], rows_v, sem).wait()
            pltpu.sync_copy(rows_v, out_hbm.at[pl.ds(off, chunk)])
            return carry

        lax.fori_loop(0, n_chunks, body, 0)

    return gather


def kernel(IX, weight):
    b0, b1 = IX.shape
    num_rows = b0 * b1
    flat_ix = IX.reshape(num_rows).astype(jnp.int32)
    out = _make_gather(num_rows)(flat_ix, weight)
    return out.reshape(b0, b1, EMB_D)


# speedup vs baseline: 1.1144x; 1.0173x over previous
import functools

import jax
import jax.numpy as jnp
from jax import lax
from jax.experimental import pallas as pl
from jax.experimental.pallas import tpu as pltpu
from jax.experimental.pallas import tpu_sc as plsc

EMB_D = 32
CH = 512
NB = 5
LOOKAHEAD = 3


def _make_gather(num_rows: int):
    info = plsc.get_sparse_core_info()
    nw = info.num_cores * info.num_subcores
    rows_per_w = num_rows // nw
    n_chunks = rows_per_w // CH
    n_groups = n_chunks // NB
    assert n_chunks % NB == 0 and n_groups >= 3
    mesh = plsc.VectorSubcoreMesh(core_axis_name="c", subcore_axis_name="s")

    @functools.partial(
        pl.kernel,
        mesh=mesh,
        out_type=jax.ShapeDtypeStruct((num_rows, EMB_D), jnp.float32),
        scratch_types=[
            pltpu.VMEM((rows_per_w,), jnp.int32),
            pltpu.VMEM((NB, CH, EMB_D), jnp.float32),
            pltpu.SemaphoreType.DMA((NB,)),
            pltpu.SemaphoreType.DMA((NB,)),
        ],
        compiler_params=pltpu.CompilerParams(use_tc_tiling_on_sc=False),
    )
    def gather(idx_hbm, table_hbm, out_hbm, idx_v, rows_v, gsem, ssem):
        wid = lax.axis_index("s") * info.num_cores + lax.axis_index("c")
        base = wid * rows_per_w
        pltpu.sync_copy(idx_hbm.at[pl.ds(base, rows_per_w)], idx_v)

        def g_desc(i, b):
            return pltpu.make_async_copy(
                table_hbm.at[idx_v.at[pl.ds(i * CH, CH)]],
                rows_v.at[b], gsem.at[b])

        def s_desc(i, b):
            return pltpu.make_async_copy(
                rows_v.at[b], out_hbm.at[pl.ds(base + i * CH, CH)],
                ssem.at[b])

        for b in range(LOOKAHEAD):
            g_desc(b, b).start()

        def step(i, b, first_group: bool, launch: bool):
            if launch:
                j = i + LOOKAHEAD
                q = (b + LOOKAHEAD) % NB
                if not (first_group and b + LOOKAHEAD < NB):
                    s_desc(j - NB, q).wait()
                g_desc(j, q).start()
            g_desc(i, b).wait()
            s_desc(i, b).start()

        for b in range(NB):
            step(b, b, first_group=True, launch=True)

        def group(g, carry):
            for b in range(NB):
                step(g * NB + b, b, first_group=False, launch=True)
            return carry

        lax.fori_loop(1, n_groups - 1, group, 0)

        last = (n_groups - 1) * NB
        for b in range(NB):
            step(last + b, b, first_group=False,
                 launch=(last + b + LOOKAHEAD < n_chunks))

        for b in range(NB):
            s_desc(n_chunks - NB + b, b).wait()

    return gather


def kernel(IX, weight):
    b0, b1 = IX.shape
    num_rows = b0 * b1
    flat_ix = IX.reshape(num_rows).astype(jnp.int32)
    out = _make_gather(num_rows)(flat_ix, weight)
    return out.reshape(b0, b1, EMB_D)


# speedup vs baseline: 1.7653x; 1.5842x over previous
import functools

import jax
import jax.numpy as jnp
from jax import lax
from jax.experimental import pallas as pl
from jax.experimental.pallas import tpu as pltpu
from jax.experimental.pallas import tpu_sc as plsc

EMB_D = 32
NB = 8
LOOKAHEAD = 4


def _make_gather(b0: int, b1: int):
    info = plsc.get_sparse_core_info()
    nw = info.num_cores * info.num_subcores
    rows_per_w = b0 // nw
    n_groups = rows_per_w // NB
    assert b0 % nw == 0 and rows_per_w % NB == 0 and n_groups >= 3
    mesh = plsc.VectorSubcoreMesh(core_axis_name="c", subcore_axis_name="s")

    @functools.partial(
        pl.kernel,
        mesh=mesh,
        out_type=jax.ShapeDtypeStruct((b0, b1, EMB_D), jnp.float32),
        scratch_types=[
            pltpu.VMEM((rows_per_w, b1), jnp.int32),
            pltpu.VMEM((NB, b1, EMB_D), jnp.float32),
            pltpu.SemaphoreType.DMA((NB,)),
            pltpu.SemaphoreType.DMA((NB,)),
        ],
        compiler_params=pltpu.CompilerParams(use_tc_tiling_on_sc=False),
    )
    def gather(idx_hbm, table_hbm, out_hbm, idx_v, rows_v, gsem, ssem):
        wid = lax.axis_index("s") * info.num_cores + lax.axis_index("c")
        base = wid * rows_per_w
        pltpu.sync_copy(idx_hbm.at[pl.ds(base, rows_per_w)], idx_v)

        def g_desc(i, b):
            return pltpu.make_async_copy(
                table_hbm.at[idx_v.at[i]], rows_v.at[b], gsem.at[b])

        def s_desc(i, b):
            return pltpu.make_async_copy(
                rows_v.at[b], out_hbm.at[base + i], ssem.at[b])

        for b in range(LOOKAHEAD):
            g_desc(b, b).start()

        def step(i, b, first_group: bool, launch: bool):
            if launch:
                j = i + LOOKAHEAD
                q = (b + LOOKAHEAD) % NB
                if not (first_group and b + LOOKAHEAD < NB):
                    s_desc(j - NB, q).wait()
                g_desc(j, q).start()
            g_desc(i, b).wait()
            s_desc(i, b).start()

        for b in range(NB):
            step(b, b, first_group=True, launch=True)

        def group(g, carry):
            for b in range(NB):
                step(g * NB + b, b, first_group=False, launch=True)
            return carry

        lax.fori_loop(1, n_groups - 1, group, 0)

        last = (n_groups - 1) * NB
        for b in range(NB):
            step(last + b, b, first_group=False,
                 launch=(last + b + LOOKAHEAD < rows_per_w))

        for b in range(NB):
            s_desc(rows_per_w - NB + b, b).wait()

    return gather


def kernel(IX, weight):
    b0, b1 = IX.shape
    return _make_gather(b0, b1)(IX.astype(jnp.int32), weight)
